# Initial kernel scaffold; baseline (speedup 1.0000x reference)
import functools

import jax
import jax.numpy as jnp
from jax import lax
from jax.experimental import pallas as pl
from jax.experimental.pallas import tpu as pltpu
from jax.experimental.pallas import tpu_sc as plsc

VOCAB_ROWS = 1002
D = 50
B, L = 4096, 256
N = B * L

NUM_CORES = 2
NUM_SUBCORES = 16
NW = NUM_CORES * NUM_SUBCORES
BPW = N // NW

GROUP = 128
CHUNK = 1024
G = CHUNK // GROUP
NCHUNK = BPW // CHUNK

_mesh = plsc.VectorSubcoreMesh(core_axis_name="c", subcore_axis_name="s")


@functools.partial(
    pl.kernel,
    mesh=_mesh,
    out_type=jax.ShapeDtypeStruct((N, D), jnp.float32),
    scratch_types=[
        pltpu.VMEM((G, GROUP), jnp.int32),
        pltpu.VMEM((CHUNK, D), jnp.float32),
        pltpu.SemaphoreType.DMA,
    ],
)
def _embed_gather(idx_hbm, table_hbm, out_hbm, idx_v, rows_v, sem):
    wid = lax.axis_index("s") * NUM_CORES + lax.axis_index("c")
    base = wid * BPW

    def chunk_body(ci, carry):
        off = base + ci * CHUNK
        row_off = off // GROUP
        pltpu.sync_copy(idx_hbm.at[pl.ds(row_off, G)], idx_v)
        copies = []
        for j in range(G):
            copies.append(
                pltpu.async_copy(
                    table_hbm.at[idx_v.at[j]],
                    rows_v.at[pl.ds(j * GROUP, GROUP)],
                    sem,
                )
            )
        for c in copies:
            c.wait()
        pltpu.sync_copy(rows_v, out_hbm.at[pl.ds(off, CHUNK)])
        return carry

    lax.fori_loop(0, NCHUNK, chunk_body, 0)


def kernel(seqs, att_mask, word_embedding):
    del att_mask
    idx2d = seqs.reshape(N // GROUP, GROUP).astype(jnp.int32)
    out = _embed_gather(idx2d, word_embedding)
    return out.reshape(B, L, D)


# speedup vs baseline: 3.6844x; 3.6844x over previous
import functools

import jax
import jax.numpy as jnp
from jax import lax
from jax.experimental import pallas as pl
from jax.experimental.pallas import tpu as pltpu
from jax.experimental.pallas import tpu_sc as plsc

VOCAB_ROWS = 1002
D = 50
DP = 64
B, L = 4096, 256
N = B * L

NUM_CORES = 2
NUM_SUBCORES = 16
NW = NUM_CORES * NUM_SUBCORES
BPW = N // NW

GROUP = 128
CHUNK = 1024
G = CHUNK // GROUP
NCHUNK = BPW // CHUNK

_mesh = plsc.VectorSubcoreMesh(core_axis_name="c", subcore_axis_name="s")


@functools.partial(
    pl.kernel,
    mesh=_mesh,
    compiler_params=pltpu.CompilerParams(use_tc_tiling_on_sc=False),
    out_type=jax.ShapeDtypeStruct((N, DP), jnp.float32),
    scratch_types=[
        pltpu.VMEM((G, GROUP), jnp.int32),
        pltpu.VMEM((CHUNK, DP), jnp.float32),
        pltpu.SemaphoreType.DMA,
    ],
)
def _embed_gather(idx_hbm, table_hbm, out_hbm, idx_v, rows_v, sem):
    wid = lax.axis_index("s") * NUM_CORES + lax.axis_index("c")
    base = wid * BPW

    def chunk_body(ci, carry):
        off = pl.multiple_of(base + ci * CHUNK, CHUNK)
        row_off = pl.multiple_of(off // GROUP, G)
        pltpu.sync_copy(idx_hbm.at[pl.ds(row_off, G)], idx_v)
        copies = []
        for j in range(G):
            copies.append(
                pltpu.async_copy(
                    table_hbm.at[idx_v.at[j]],
                    rows_v.at[pl.ds(j * GROUP, GROUP)],
                    sem,
                )
            )
        for c in copies:
            c.wait()
        pltpu.sync_copy(rows_v, out_hbm.at[pl.ds(off, CHUNK)])
        return carry

    lax.fori_loop(0, NCHUNK, chunk_body, 0)


def kernel(seqs, att_mask, word_embedding):
    del att_mask
    idx2d = seqs.reshape(N // GROUP, GROUP).astype(jnp.int32)
    table_p = jnp.pad(word_embedding, ((0, 0), (0, DP - D)))
    out = _embed_gather(idx2d, table_p)
    return out[:, :D].reshape(B, L, D)
